# baseline (speedup 1.0000x reference)
import functools

import jax
import jax.numpy as jnp
from jax import lax
from jax.experimental import pallas as pl
from jax.experimental.pallas import tpu as pltpu
from jax.experimental.pallas import tpu_sc as plsc

N = 4096
E = 16384
B = 128
DIM = 64
NC = 2
NS = 16
NW = NC * NS
EB = 512
CHUNK = 128

_f32 = jnp.float32


def _lrelu(x):
    return jnp.where(x >= 0, x, 0.01 * x)


def _mesh():
    return plsc.VectorSubcoreMesh(
        core_axis_name="c", subcore_axis_name="s",
        num_cores=NC, num_subcores=NS)


def _sc_gather(table, idx3, V, D, Btot, chunk):
    per_w = Btot // NW
    k = per_w // chunk

    @functools.partial(
        pl.kernel, mesh=_mesh(),
        out_type=jax.ShapeDtypeStruct((Btot, D), _f32),
        compiler_params=pltpu.CompilerParams(use_tc_tiling_on_sc=False),
        scratch_types=[
            pltpu.VMEM((k, chunk), jnp.int32),
            pltpu.VMEM((per_w, D), _f32),
            pltpu.SemaphoreType.DMA,
        ],
    )
    def gather_k(table_hbm, idx_hbm, out_hbm, idx_v, rows_v, sem):
        wid = lax.axis_index("s") * NC + lax.axis_index("c")
        pltpu.sync_copy(idx_hbm.at[wid], idx_v)
        cps = [
            pltpu.async_copy(table_hbm.at[idx_v.at[j]],
                             rows_v.at[pl.ds(j * chunk, chunk)], sem)
            for j in range(k)
        ]
        for c in cps:
            c.wait()
        pltpu.sync_copy(rows_v, out_hbm.at[pl.ds(wid * per_w, per_w)])

    return gather_k(table, idx3)


def _sc_scatter(msg, idx3, zeros_n):
    per_w = E // NW
    k = per_w // CHUNK
    rpt = N // NS

    @functools.partial(
        pl.kernel, mesh=_mesh(),
        out_type=jax.ShapeDtypeStruct((NC, N, DIM), _f32),
        compiler_params=pltpu.CompilerParams(use_tc_tiling_on_sc=False),
        scratch_types=[
            pltpu.VMEM((k, CHUNK), jnp.int32),
            pltpu.VMEM((per_w, DIM), _f32),
            pltpu.VMEM_SHARED((N, DIM), _f32),
        ],
    )
    def scatter_k(msg_hbm, idx_hbm, z_hbm, out_hbm, idx_v, msg_v, acc_sh):
        cid = lax.axis_index("c")
        sid = lax.axis_index("s")
        wid = sid * NC + cid
        pltpu.sync_copy(z_hbm.at[pl.ds(sid * rpt, rpt)],
                        acc_sh.at[pl.ds(sid * rpt, rpt)])
        pltpu.sync_copy(idx_hbm.at[wid], idx_v)
        pltpu.sync_copy(msg_hbm.at[pl.ds(wid * per_w, per_w)], msg_v)
        plsc.subcore_barrier()
        for j in range(k):
            pltpu.sync_copy(msg_v.at[pl.ds(j * CHUNK, CHUNK)],
                            acc_sh.at[idx_v.at[j]], add=True)
        plsc.subcore_barrier()
        pltpu.sync_copy(acc_sh.at[pl.ds(sid * rpt, rpt)],
                        out_hbm.at[cid, pl.ds(sid * rpt, rpt)])

    return scatter_k(msg, idx3, zeros_n)


def _tc_pre(x, lin0_w, lin0_b, edge_attr, en_w1, b1c):
    def body(x_r, w_r, b_r, ea_r, w1_r, b1_r, out0_r, hT_r):
        out0_r[...] = _lrelu(
            jnp.dot(x_r[...], w_r[...], preferred_element_type=_f32) + b_r[...])
        hT_r[...] = _lrelu(
            lax.dot_general(w1_r[...], ea_r[...], (((0,), (1,)), ((), ())),
                            preferred_element_type=_f32) + b1_r[...])

    return pl.pallas_call(
        body,
        out_shape=(jax.ShapeDtypeStruct((N, DIM), _f32),
                   jax.ShapeDtypeStruct((128, E), _f32)),
    )(x, lin0_w, lin0_b, edge_attr, en_w1, b1c)


def _tc_msg(hT, osrc, en_w2, b2T):
    def body(hT_r, o_r, w2_r, b2T_r, msg_r):
        oT = o_r[...].T
        ewT = lax.dot_general(w2_r[...], hT_r[...], (((0,), (0,)), ((), ())),
                              preferred_element_type=_f32)
        accT = jnp.dot(b2T_r[...], oT,
                       preferred_element_type=_f32)
        for d in range(DIM):
            accT += oT[d:d + 1, :] * ewT[d * DIM:(d + 1) * DIM, :]
        msg_r[...] = accT.T

    grid = E // EB
    return pl.pallas_call(
        body,
        grid=(grid,),
        in_specs=[
            pl.BlockSpec((128, EB), lambda i: (0, i)),
            pl.BlockSpec((EB, DIM), lambda i: (i, 0)),
            pl.BlockSpec((128, DIM * DIM), lambda i: (0, 0)),
            pl.BlockSpec((DIM, DIM), lambda i: (0, 0)),
        ],
        out_specs=pl.BlockSpec((EB, DIM), lambda i: (i, 0)),
        out_shape=jax.ShapeDtypeStruct((E, DIM), _f32),
    )(hT, osrc, en_w2, b2T)


def _tc_update(aggp, onesp, h, conv_root, conv_bias,
               w_ir, w_iz, w_in, w_hr, w_hz, w_hn, b_r, b_z, b_in, b_hn):
    def body(aggp_r, onesp_r, h_r, cr_r, cb_r,
             wir_r, wiz_r, win_r, whr_r, whz_r, whn_r,
             br_r, bz_r, bin_r, bhn_r, out_r):
        h0 = h_r[...]
        cnt = onesp_r[0, :, 0:1] + onesp_r[1, :, 0:1]
        inv = 1.0 / jnp.clip(cnt, 1.0, None)
        agg = (aggp_r[0] + aggp_r[1]) * inv
        m = _lrelu(agg + jnp.dot(h0, cr_r[...], preferred_element_type=_f32)
                   + cb_r[...])
        r = jax.nn.sigmoid(jnp.dot(m, wir_r[...], preferred_element_type=_f32)
                           + jnp.dot(h0, whr_r[...], preferred_element_type=_f32)
                           + br_r[...])
        z = jax.nn.sigmoid(jnp.dot(m, wiz_r[...], preferred_element_type=_f32)
                           + jnp.dot(h0, whz_r[...], preferred_element_type=_f32)
                           + bz_r[...])
        hn = jnp.dot(h0, whn_r[...], preferred_element_type=_f32) + bhn_r[...]
        n = jnp.tanh(jnp.dot(m, win_r[...], preferred_element_type=_f32)
                     + bin_r[...] + r * hn)
        out_r[...] = (1.0 - z) * n + z * h0

    return pl.pallas_call(
        body,
        out_shape=jax.ShapeDtypeStruct((N, DIM), _f32),
    )(aggp, onesp, h, conv_root, conv_bias,
      w_ir, w_iz, w_in, w_hr, w_hz, w_hn, b_r, b_z, b_in, b_hn)


def _tc_post(h, lin1_w, lin1_b, batch2d,
             wq_i, wq_f, wq_g, wq_o, wh_i, wh_f, wh_g, wh_o,
             b_i, b_f, b_g, b_o, lin3_w, lin3_b,
             n2s_w1, n2s_b1, n2s_w2, n2s_b2,
             n2j_w1, n2j_b1, n2j_w2, n2j_b2,
             stem2d, jb2d, pairmat):
    def body(h_r, w1_r, b1_r, bat_r,
             wqi_r, wqf_r, wqg_r, wqo_r, whi_r, whf_r, whg_r, who_r,
             bi_r, bf_r, bg_r, bo_r, w3_r, b3_r,
             sw1_r, sb1_r, sw2_r, sb2_r, jw1_r, jb1_r, jw2_r, jb2_r,
             sidx_r, jidx_r, pm_r, sout_r, stem_r, jbm_r):
        h0 = h_r[...]
        pa = _lrelu(
            jnp.dot(h0, w1_r[...], preferred_element_type=_f32) + b1_r[...])
        sh_all = _lrelu(
            jnp.dot(pa, sw1_r[...], preferred_element_type=_f32) + sb1_r[...])
        jh_all = _lrelu(
            jnp.dot(pa, jw1_r[...], preferred_element_type=_f32) + jb1_r[...])
        iota_n = lax.broadcasted_iota(jnp.int32, (256, N), 1)
        oh_s = (sidx_r[...] == iota_n).astype(_f32)
        oh_j = (jidx_r[...] == iota_n).astype(_f32)
        sh_g = jnp.dot(oh_s, sh_all, preferred_element_type=_f32)
        stem_r[...] = (jnp.dot(sh_g, sw2_r[...], preferred_element_type=_f32)
                       + sb2_r[...])
        jh_g = jnp.dot(oh_j, jh_all, preferred_element_type=_f32)
        jb = jnp.dot(jh_g, jw2_r[...], preferred_element_type=_f32) + jb2_r[...]
        jbm_r[...] = jnp.dot(pm_r[...], jb, preferred_element_type=_f32)
        ids = lax.broadcasted_iota(jnp.int32, (N, B), 1)
        mb = bat_r[...] == ids
        mm = mb.astype(_f32)
        q_star = jnp.zeros((B, 2 * DIM), _f32)
        hs = jnp.zeros((B, DIM), _f32)
        cs = jnp.zeros((B, DIM), _f32)
        for _ in range(3):
            g_i = (jnp.dot(q_star, wqi_r[...], preferred_element_type=_f32)
                   + jnp.dot(hs, whi_r[...], preferred_element_type=_f32)
                   + bi_r[...])
            g_f = (jnp.dot(q_star, wqf_r[...], preferred_element_type=_f32)
                   + jnp.dot(hs, whf_r[...], preferred_element_type=_f32)
                   + bf_r[...])
            g_g = (jnp.dot(q_star, wqg_r[...], preferred_element_type=_f32)
                   + jnp.dot(hs, whg_r[...], preferred_element_type=_f32)
                   + bg_r[...])
            g_o = (jnp.dot(q_star, wqo_r[...], preferred_element_type=_f32)
                   + jnp.dot(hs, who_r[...], preferred_element_type=_f32)
                   + bo_r[...])
            cs = jax.nn.sigmoid(g_f) * cs + jax.nn.sigmoid(g_i) * jnp.tanh(g_g)
            hs = jax.nn.sigmoid(g_o) * jnp.tanh(cs)
            qb = jnp.dot(mm, hs, preferred_element_type=_f32)
            e = jnp.sum(h0 * qb, axis=1, keepdims=True)
            masked = jnp.where(mb, e, -jnp.inf)
            emax = jnp.max(masked, axis=0, keepdims=True)
            emax = jnp.where(jnp.isfinite(emax), emax, 0.0)
            eb = jnp.sum(mm * emax, axis=1, keepdims=True)
            ee = jnp.exp(e - eb)
            denom = lax.dot_general(mm, ee, (((0,), (0,)), ((), ())),
                                    preferred_element_type=_f32)
            db = jnp.dot(mm, denom, preferred_element_type=_f32)
            a = ee / db
            r_read = lax.dot_general(mm, a * h0, (((0,), (0,)), ((), ())),
                                     preferred_element_type=_f32)
            q_star = jnp.concatenate([hs, r_read], axis=1)
        sout_r[...] = (jnp.dot(q_star, w3_r[...], preferred_element_type=_f32)
                       + b3_r[...])

    return pl.pallas_call(
        body,
        out_shape=(jax.ShapeDtypeStruct((B, 2), _f32),
                   jax.ShapeDtypeStruct((256, 105), _f32),
                   jax.ShapeDtypeStruct((B, 1), _f32)),
    )(h, lin1_w, lin1_b, batch2d,
      wq_i, wq_f, wq_g, wq_o, wh_i, wh_f, wh_g, wh_o,
      b_i, b_f, b_g, b_o, lin3_w, lin3_b,
      n2s_w1, n2s_b1, n2s_w2, n2s_b2, n2j_w1, n2j_b1, n2j_w2, n2j_b2,
      stem2d, jb2d, pairmat)


def kernel(x, edge_attr, lin0_w, lin0_b, en_w1, en_b1, en_w2, en_b2,
           conv_root, conv_bias, gru_w_ih, gru_w_hh, gru_b_ih, gru_b_hh,
           lin1_w, lin1_b, lstm_w_ih, lstm_w_hh, lstm_b_ih, lstm_b_hh,
           lin3_w, lin3_b, n2s_w1, n2s_b1, n2s_w2, n2s_b2,
           n2j_w1, n2j_b1, n2j_w2, n2j_b2,
           edge_index, batch, stem_atmidx, jbond_atmidx):
    i32 = jnp.int32
    src3 = edge_index[0].astype(i32).reshape(NW, E // NW // CHUNK, CHUNK)
    dst3 = edge_index[1].astype(i32).reshape(NW, E // NW // CHUNK, CHUNK)
    b1c = en_b1.reshape(128, 1)
    b2T = en_b2.reshape(DIM, DIM).T
    zeros_n = jnp.zeros((N, DIM), _f32)
    ones_e = jnp.ones((E, DIM), _f32)

    w_ir, w_iz, w_in = (gru_w_ih[0:DIM].T, gru_w_ih[DIM:2 * DIM].T,
                        gru_w_ih[2 * DIM:3 * DIM].T)
    w_hr, w_hz, w_hn = (gru_w_hh[0:DIM].T, gru_w_hh[DIM:2 * DIM].T,
                        gru_w_hh[2 * DIM:3 * DIM].T)
    b_r = (gru_b_ih[0:DIM] + gru_b_hh[0:DIM]).reshape(1, DIM)
    b_z = (gru_b_ih[DIM:2 * DIM] + gru_b_hh[DIM:2 * DIM]).reshape(1, DIM)
    b_in = gru_b_ih[2 * DIM:3 * DIM].reshape(1, DIM)
    b_hn = gru_b_hh[2 * DIM:3 * DIM].reshape(1, DIM)

    wq_i, wq_f, wq_g, wq_o = (lstm_w_ih[0:DIM].T, lstm_w_ih[DIM:2 * DIM].T,
                              lstm_w_ih[2 * DIM:3 * DIM].T,
                              lstm_w_ih[3 * DIM:4 * DIM].T)
    wh_i, wh_f, wh_g, wh_o = (lstm_w_hh[0:DIM].T, lstm_w_hh[DIM:2 * DIM].T,
                              lstm_w_hh[2 * DIM:3 * DIM].T,
                              lstm_w_hh[3 * DIM:4 * DIM].T)
    lb = lstm_b_ih + lstm_b_hh
    b_i, b_f = lb[0:DIM].reshape(1, DIM), lb[DIM:2 * DIM].reshape(1, DIM)
    b_g, b_o = (lb[2 * DIM:3 * DIM].reshape(1, DIM),
                lb[3 * DIM:4 * DIM].reshape(1, DIM))

    out0, hT = _tc_pre(x, lin0_w, lin0_b.reshape(1, DIM), edge_attr, en_w1,
                       b1c)
    onesp = _sc_scatter(ones_e, dst3, zeros_n)

    h = out0
    for _ in range(3):
        osrc = _sc_gather(h, src3, N, DIM, E, CHUNK)
        msg = _tc_msg(hT, osrc, en_w2, b2T)
        aggp = _sc_scatter(msg, dst3, zeros_n)
        h = _tc_update(aggp, onesp, h, conv_root, conv_bias.reshape(1, DIM),
                       w_ir, w_iz, w_in, w_hr, w_hz, w_hn,
                       b_r, b_z, b_in, b_hn)

    cols = jnp.arange(256, dtype=i32)
    pairmat = (cols[None, :] // 2
               == jnp.arange(B, dtype=i32)[:, None]).astype(_f32) * 0.5

    sout, stem_preds, jbm = _tc_post(
        h, lin1_w, lin1_b.reshape(1, 8 * DIM), batch.astype(i32).reshape(N, 1),
        wq_i, wq_f, wq_g, wq_o, wh_i, wh_f, wh_g, wh_o,
        b_i, b_f, b_g, b_o, lin3_w, lin3_b.reshape(1, 2),
        n2s_w1, n2s_b1.reshape(1, DIM), n2s_w2, n2s_b2.reshape(1, 105),
        n2j_w1, n2j_b1.reshape(1, DIM), n2j_w2, n2j_b2.reshape(1, 1),
        stem_atmidx.astype(i32).reshape(256, 1),
        jbond_atmidx.astype(i32).reshape(256, 1), pairmat)

    return (sout, stem_preds, jbm.reshape(B))


# speedup vs baseline: 1.0551x; 1.0551x over previous
import functools

import jax
import jax.numpy as jnp
from jax import lax
from jax.experimental import pallas as pl
from jax.experimental.pallas import tpu as pltpu
from jax.experimental.pallas import tpu_sc as plsc

N = 4096
E = 16384
B = 128
DIM = 64
NC = 2
NS = 16
NW = NC * NS
EB = 512
CHUNK = 128

_f32 = jnp.float32


def _lrelu(x):
    return jnp.where(x >= 0, x, 0.01 * x)


def _mesh():
    return plsc.VectorSubcoreMesh(
        core_axis_name="c", subcore_axis_name="s",
        num_cores=NC, num_subcores=NS)


def _sc_gather(table, idx3, V, D, Btot, chunk):
    per_w = Btot // NW
    k = per_w // chunk

    @functools.partial(
        pl.kernel, mesh=_mesh(),
        out_type=jax.ShapeDtypeStruct((Btot, D), _f32),
        compiler_params=pltpu.CompilerParams(use_tc_tiling_on_sc=False),
        scratch_types=[
            pltpu.VMEM((k, chunk), jnp.int32),
            pltpu.VMEM((per_w, D), _f32),
            pltpu.SemaphoreType.DMA,
        ],
    )
    def gather_k(table_hbm, idx_hbm, out_hbm, idx_v, rows_v, sem):
        wid = lax.axis_index("s") * NC + lax.axis_index("c")
        pltpu.sync_copy(idx_hbm.at[wid], idx_v)
        cps = [
            pltpu.async_copy(table_hbm.at[idx_v.at[j]],
                             rows_v.at[pl.ds(j * chunk, chunk)], sem)
            for j in range(k)
        ]
        for c in cps:
            c.wait()
        pltpu.sync_copy(rows_v, out_hbm.at[pl.ds(wid * per_w, per_w)])

    return gather_k(table, idx3)


def _sc_scatter(msg, idx3, zeros_n):
    per_w = E // NW
    k = per_w // CHUNK
    rpt = N // NS

    @functools.partial(
        pl.kernel, mesh=_mesh(),
        out_type=jax.ShapeDtypeStruct((NC, N, DIM), _f32),
        compiler_params=pltpu.CompilerParams(use_tc_tiling_on_sc=False),
        scratch_types=[
            pltpu.VMEM((k, CHUNK), jnp.int32),
            pltpu.VMEM((per_w, DIM), _f32),
            pltpu.VMEM_SHARED((N, DIM), _f32),
        ],
    )
    def scatter_k(msg_hbm, idx_hbm, z_hbm, out_hbm, idx_v, msg_v, acc_sh):
        cid = lax.axis_index("c")
        sid = lax.axis_index("s")
        wid = sid * NC + cid
        pltpu.sync_copy(z_hbm.at[pl.ds(sid * rpt, rpt)],
                        acc_sh.at[pl.ds(sid * rpt, rpt)])
        pltpu.sync_copy(idx_hbm.at[wid], idx_v)
        pltpu.sync_copy(msg_hbm.at[pl.ds(wid * per_w, per_w)], msg_v)
        plsc.subcore_barrier()
        for j in range(k):
            pltpu.sync_copy(msg_v.at[pl.ds(j * CHUNK, CHUNK)],
                            acc_sh.at[idx_v.at[j]], add=True)
        plsc.subcore_barrier()
        pltpu.sync_copy(acc_sh.at[pl.ds(sid * rpt, rpt)],
                        out_hbm.at[cid, pl.ds(sid * rpt, rpt)])

    return scatter_k(msg, idx3, zeros_n)


def _tc_pre(x, lin0_w, lin0_b, edge_attr, en_w1, b1c):
    def body(x_r, w_r, b_r, ea_r, w1_r, b1_r, out0_r, hT_r):
        out0_r[...] = _lrelu(
            jnp.dot(x_r[...], w_r[...], preferred_element_type=_f32) + b_r[...])
        hT_r[...] = _lrelu(
            lax.dot_general(w1_r[...], ea_r[...], (((0,), (1,)), ((), ())),
                            preferred_element_type=_f32) + b1_r[...])

    return pl.pallas_call(
        body,
        out_shape=(jax.ShapeDtypeStruct((N, DIM), _f32),
                   jax.ShapeDtypeStruct((128, E), _f32)),
    )(x, lin0_w, lin0_b, edge_attr, en_w1, b1c)


def _tc_msg(hT, osrc, w2T, b2T):
    def body(hT_r, o_r, w2T_r, b2T_r, msg_r):
        oT = o_r[...].T
        ewT = jnp.dot(w2T_r[...], hT_r[...],
                      preferred_element_type=_f32)
        accT = jnp.dot(b2T_r[...], oT,
                       preferred_element_type=_f32)
        for d in range(DIM):
            accT += oT[d:d + 1, :] * ewT[d * DIM:(d + 1) * DIM, :]
        msg_r[...] = accT.T

    grid = E // EB
    return pl.pallas_call(
        body,
        grid=(grid,),
        in_specs=[
            pl.BlockSpec((128, EB), lambda i: (0, i)),
            pl.BlockSpec((EB, DIM), lambda i: (i, 0)),
            pl.BlockSpec((DIM * DIM, 128), lambda i: (0, 0)),
            pl.BlockSpec((DIM, DIM), lambda i: (0, 0)),
        ],
        out_specs=pl.BlockSpec((EB, DIM), lambda i: (i, 0)),
        out_shape=jax.ShapeDtypeStruct((E, DIM), _f32),
    )(hT, osrc, w2T, b2T)


def _tc_update(aggp, onesp, h, conv_root, conv_bias,
               w_ir, w_iz, w_in, w_hr, w_hz, w_hn, b_r, b_z, b_in, b_hn):
    def body(aggp_r, onesp_r, h_r, cr_r, cb_r,
             wir_r, wiz_r, win_r, whr_r, whz_r, whn_r,
             br_r, bz_r, bin_r, bhn_r, out_r):
        h0 = h_r[...]
        cnt = onesp_r[0, :, 0:1] + onesp_r[1, :, 0:1]
        inv = 1.0 / jnp.clip(cnt, 1.0, None)
        agg = (aggp_r[0] + aggp_r[1]) * inv
        m = _lrelu(agg + jnp.dot(h0, cr_r[...], preferred_element_type=_f32)
                   + cb_r[...])
        r = jax.nn.sigmoid(jnp.dot(m, wir_r[...], preferred_element_type=_f32)
                           + jnp.dot(h0, whr_r[...], preferred_element_type=_f32)
                           + br_r[...])
        z = jax.nn.sigmoid(jnp.dot(m, wiz_r[...], preferred_element_type=_f32)
                           + jnp.dot(h0, whz_r[...], preferred_element_type=_f32)
                           + bz_r[...])
        hn = jnp.dot(h0, whn_r[...], preferred_element_type=_f32) + bhn_r[...]
        n = jnp.tanh(jnp.dot(m, win_r[...], preferred_element_type=_f32)
                     + bin_r[...] + r * hn)
        out_r[...] = (1.0 - z) * n + z * h0

    return pl.pallas_call(
        body,
        out_shape=jax.ShapeDtypeStruct((N, DIM), _f32),
    )(aggp, onesp, h, conv_root, conv_bias,
      w_ir, w_iz, w_in, w_hr, w_hz, w_hn, b_r, b_z, b_in, b_hn)


def _tc_post(h, lin1_w, lin1_b, batch2d,
             wq_i, wq_f, wq_g, wq_o, wh_i, wh_f, wh_g, wh_o,
             b_i, b_f, b_g, b_o, lin3_w, lin3_b,
             n2s_w1, n2s_b1, n2s_w2, n2s_b2,
             n2j_w1, n2j_b1, n2j_w2, n2j_b2,
             stem2d, jb2d, pairmat):
    def body(h_r, w1_r, b1_r, bat_r,
             wqi_r, wqf_r, wqg_r, wqo_r, whi_r, whf_r, whg_r, who_r,
             bi_r, bf_r, bg_r, bo_r, w3_r, b3_r,
             sw1_r, sb1_r, sw2_r, sb2_r, jw1_r, jb1_r, jw2_r, jb2_r,
             sidx_r, jidx_r, pm_r, sout_r, stem_r, jbm_r):
        h0 = h_r[...]
        pa = _lrelu(
            jnp.dot(h0, w1_r[...], preferred_element_type=_f32) + b1_r[...])
        sh_all = _lrelu(
            jnp.dot(pa, sw1_r[...], preferred_element_type=_f32) + sb1_r[...])
        jh_all = _lrelu(
            jnp.dot(pa, jw1_r[...], preferred_element_type=_f32) + jb1_r[...])
        iota_n = lax.broadcasted_iota(jnp.int32, (256, N), 1)
        oh_s = (sidx_r[...] == iota_n).astype(_f32)
        oh_j = (jidx_r[...] == iota_n).astype(_f32)
        sh_g = jnp.dot(oh_s, sh_all, preferred_element_type=_f32)
        stem_r[...] = (jnp.dot(sh_g, sw2_r[...], preferred_element_type=_f32)
                       + sb2_r[...])
        jh_g = jnp.dot(oh_j, jh_all, preferred_element_type=_f32)
        jb = jnp.dot(jh_g, jw2_r[...], preferred_element_type=_f32) + jb2_r[...]
        jbm_r[...] = jnp.dot(pm_r[...], jb, preferred_element_type=_f32)
        ids = lax.broadcasted_iota(jnp.int32, (N, B), 1)
        mb = bat_r[...] == ids
        mm = mb.astype(_f32)
        q_star = jnp.zeros((B, 2 * DIM), _f32)
        hs = jnp.zeros((B, DIM), _f32)
        cs = jnp.zeros((B, DIM), _f32)
        for _ in range(3):
            g_i = (jnp.dot(q_star, wqi_r[...], preferred_element_type=_f32)
                   + jnp.dot(hs, whi_r[...], preferred_element_type=_f32)
                   + bi_r[...])
            g_f = (jnp.dot(q_star, wqf_r[...], preferred_element_type=_f32)
                   + jnp.dot(hs, whf_r[...], preferred_element_type=_f32)
                   + bf_r[...])
            g_g = (jnp.dot(q_star, wqg_r[...], preferred_element_type=_f32)
                   + jnp.dot(hs, whg_r[...], preferred_element_type=_f32)
                   + bg_r[...])
            g_o = (jnp.dot(q_star, wqo_r[...], preferred_element_type=_f32)
                   + jnp.dot(hs, who_r[...], preferred_element_type=_f32)
                   + bo_r[...])
            cs = jax.nn.sigmoid(g_f) * cs + jax.nn.sigmoid(g_i) * jnp.tanh(g_g)
            hs = jax.nn.sigmoid(g_o) * jnp.tanh(cs)
            qb = jnp.dot(mm, hs, preferred_element_type=_f32)
            e = jnp.sum(h0 * qb, axis=1, keepdims=True)
            masked = jnp.where(mb, e, -jnp.inf)
            emax = jnp.max(masked, axis=0, keepdims=True)
            emax = jnp.where(jnp.isfinite(emax), emax, 0.0)
            eb = jnp.sum(mm * emax, axis=1, keepdims=True)
            ee = jnp.exp(e - eb)
            denom = lax.dot_general(mm, ee, (((0,), (0,)), ((), ())),
                                    preferred_element_type=_f32)
            db = jnp.dot(mm, denom, preferred_element_type=_f32)
            a = ee / db
            r_read = lax.dot_general(mm, a * h0, (((0,), (0,)), ((), ())),
                                     preferred_element_type=_f32)
            q_star = jnp.concatenate([hs, r_read], axis=1)
        sout_r[...] = (jnp.dot(q_star, w3_r[...], preferred_element_type=_f32)
                       + b3_r[...])

    return pl.pallas_call(
        body,
        out_shape=(jax.ShapeDtypeStruct((B, 2), _f32),
                   jax.ShapeDtypeStruct((256, 105), _f32),
                   jax.ShapeDtypeStruct((B, 1), _f32)),
    )(h, lin1_w, lin1_b, batch2d,
      wq_i, wq_f, wq_g, wq_o, wh_i, wh_f, wh_g, wh_o,
      b_i, b_f, b_g, b_o, lin3_w, lin3_b,
      n2s_w1, n2s_b1, n2s_w2, n2s_b2, n2j_w1, n2j_b1, n2j_w2, n2j_b2,
      stem2d, jb2d, pairmat)


def kernel(x, edge_attr, lin0_w, lin0_b, en_w1, en_b1, en_w2, en_b2,
           conv_root, conv_bias, gru_w_ih, gru_w_hh, gru_b_ih, gru_b_hh,
           lin1_w, lin1_b, lstm_w_ih, lstm_w_hh, lstm_b_ih, lstm_b_hh,
           lin3_w, lin3_b, n2s_w1, n2s_b1, n2s_w2, n2s_b2,
           n2j_w1, n2j_b1, n2j_w2, n2j_b2,
           edge_index, batch, stem_atmidx, jbond_atmidx):
    i32 = jnp.int32
    src3 = edge_index[0].astype(i32).reshape(NW, E // NW // CHUNK, CHUNK)
    dst3 = edge_index[1].astype(i32).reshape(NW, E // NW // CHUNK, CHUNK)
    b1c = en_b1.reshape(128, 1)
    w2T = en_w2.T
    b2T = en_b2.reshape(DIM, DIM).T
    zeros_n = jnp.zeros((N, DIM), _f32)
    ones_e = jnp.ones((E, DIM), _f32)

    w_ir, w_iz, w_in = (gru_w_ih[0:DIM].T, gru_w_ih[DIM:2 * DIM].T,
                        gru_w_ih[2 * DIM:3 * DIM].T)
    w_hr, w_hz, w_hn = (gru_w_hh[0:DIM].T, gru_w_hh[DIM:2 * DIM].T,
                        gru_w_hh[2 * DIM:3 * DIM].T)
    b_r = (gru_b_ih[0:DIM] + gru_b_hh[0:DIM]).reshape(1, DIM)
    b_z = (gru_b_ih[DIM:2 * DIM] + gru_b_hh[DIM:2 * DIM]).reshape(1, DIM)
    b_in = gru_b_ih[2 * DIM:3 * DIM].reshape(1, DIM)
    b_hn = gru_b_hh[2 * DIM:3 * DIM].reshape(1, DIM)

    wq_i, wq_f, wq_g, wq_o = (lstm_w_ih[0:DIM].T, lstm_w_ih[DIM:2 * DIM].T,
                              lstm_w_ih[2 * DIM:3 * DIM].T,
                              lstm_w_ih[3 * DIM:4 * DIM].T)
    wh_i, wh_f, wh_g, wh_o = (lstm_w_hh[0:DIM].T, lstm_w_hh[DIM:2 * DIM].T,
                              lstm_w_hh[2 * DIM:3 * DIM].T,
                              lstm_w_hh[3 * DIM:4 * DIM].T)
    lb = lstm_b_ih + lstm_b_hh
    b_i, b_f = lb[0:DIM].reshape(1, DIM), lb[DIM:2 * DIM].reshape(1, DIM)
    b_g, b_o = (lb[2 * DIM:3 * DIM].reshape(1, DIM),
                lb[3 * DIM:4 * DIM].reshape(1, DIM))

    out0, hT = _tc_pre(x, lin0_w, lin0_b.reshape(1, DIM), edge_attr, en_w1,
                       b1c)
    onesp = _sc_scatter(ones_e, dst3, zeros_n)

    h = out0
    for _ in range(3):
        osrc = _sc_gather(h, src3, N, DIM, E, CHUNK)
        msg = _tc_msg(hT, osrc, w2T, b2T)
        aggp = _sc_scatter(msg, dst3, zeros_n)
        h = _tc_update(aggp, onesp, h, conv_root, conv_bias.reshape(1, DIM),
                       w_ir, w_iz, w_in, w_hr, w_hz, w_hn,
                       b_r, b_z, b_in, b_hn)

    cols = jnp.arange(256, dtype=i32)
    pairmat = (cols[None, :] // 2
               == jnp.arange(B, dtype=i32)[:, None]).astype(_f32) * 0.5

    sout, stem_preds, jbm = _tc_post(
        h, lin1_w, lin1_b.reshape(1, 8 * DIM), batch.astype(i32).reshape(N, 1),
        wq_i, wq_f, wq_g, wq_o, wh_i, wh_f, wh_g, wh_o,
        b_i, b_f, b_g, b_o, lin3_w, lin3_b.reshape(1, 2),
        n2s_w1, n2s_b1.reshape(1, DIM), n2s_w2, n2s_b2.reshape(1, 105),
        n2j_w1, n2j_b1.reshape(1, DIM), n2j_w2, n2j_b2.reshape(1, 1),
        stem_atmidx.astype(i32).reshape(256, 1),
        jbond_atmidx.astype(i32).reshape(256, 1), pairmat)

    return (sout, stem_preds, jbm.reshape(B))


# speedup vs baseline: 1.1946x; 1.1322x over previous
import functools

import jax
import jax.numpy as jnp
from jax import lax
from jax.experimental import pallas as pl
from jax.experimental.pallas import tpu as pltpu
from jax.experimental.pallas import tpu_sc as plsc

N = 4096
E = 16384
B = 128
DIM = 64
W = 128
NC = 2
NS = 16
NW = NC * NS
EB = 512
CHUNK = 128
KCH = E // NW // CHUNK

_f32 = jnp.float32


def _lrelu(x):
    return jnp.where(x >= 0, x, 0.01 * x)


def _mesh():
    return plsc.VectorSubcoreMesh(
        core_axis_name="c", subcore_axis_name="s",
        num_cores=NC, num_subcores=NS)


def _sc_gather(table, idx2):
    per_w = E // NW

    @functools.partial(
        pl.kernel, mesh=_mesh(),
        out_type=jax.ShapeDtypeStruct((E, W), _f32),
        compiler_params=pltpu.CompilerParams(use_tc_tiling_on_sc=False),
        scratch_types=[
            pltpu.VMEM((KCH, CHUNK), jnp.int32),
            pltpu.VMEM((per_w, W), _f32),
            pltpu.SemaphoreType.DMA,
        ],
    )
    def gather_k(table_hbm, idx_hbm, out_hbm, idx_v, rows_v, sem):
        wid = lax.axis_index("s") * NC + lax.axis_index("c")
        pltpu.sync_copy(idx_hbm.at[pl.ds(wid * KCH, KCH)], idx_v)
        cps = [
            pltpu.async_copy(table_hbm.at[idx_v.at[j]],
                             rows_v.at[pl.ds(j * CHUNK, CHUNK)], sem)
            for j in range(KCH)
        ]
        for c in cps:
            c.wait()
        pltpu.sync_copy(rows_v, out_hbm.at[pl.ds(wid * per_w, per_w)])

    return gather_k(table, idx2)


def _sc_scatter(msg, idx2, zeros_n):
    per_w = E // NW
    rpt = N // NS

    @functools.partial(
        pl.kernel, mesh=_mesh(),
        out_type=jax.ShapeDtypeStruct((NC, N, W), _f32),
        compiler_params=pltpu.CompilerParams(use_tc_tiling_on_sc=False),
        scratch_types=[
            pltpu.VMEM((KCH, CHUNK), jnp.int32),
            pltpu.VMEM((per_w, W), _f32),
            pltpu.VMEM_SHARED((N, W), _f32),
        ],
    )
    def scatter_k(msg_hbm, idx_hbm, z_hbm, out_hbm, idx_v, msg_v, acc_sh):
        cid = lax.axis_index("c")
        sid = lax.axis_index("s")
        wid = sid * NC + cid
        pltpu.sync_copy(z_hbm.at[pl.ds(sid * rpt, rpt)],
                        acc_sh.at[pl.ds(sid * rpt, rpt)])
        pltpu.sync_copy(idx_hbm.at[pl.ds(wid * KCH, KCH)], idx_v)
        pltpu.sync_copy(msg_hbm.at[pl.ds(wid * per_w, per_w)], msg_v)
        plsc.subcore_barrier()
        for j in range(KCH):
            pltpu.sync_copy(msg_v.at[pl.ds(j * CHUNK, CHUNK)],
                            acc_sh.at[idx_v.at[j]], add=True)
        plsc.subcore_barrier()
        pltpu.sync_copy(acc_sh.at[pl.ds(sid * rpt, rpt)],
                        out_hbm.at[cid, pl.ds(sid * rpt, rpt)])

    return scatter_k(msg, idx2, zeros_n)


def _tc_pre(x, lin0_w, lin0_b):
    def body(x_r, w_r, b_r, out0_r):
        out0_r[:, 0:DIM] = _lrelu(
            jnp.dot(x_r[...], w_r[...], preferred_element_type=_f32) + b_r[...])
        out0_r[:, DIM:W] = jnp.zeros((N, W - DIM), _f32)

    return pl.pallas_call(
        body,
        out_shape=jax.ShapeDtypeStruct((N, W), _f32),
    )(x, lin0_w, lin0_b)


def _tc_msg(eaT, osrc, w1T, b1c, w2T, b2T):
    def body(ea_r, o_r, w1T_r, b1_r, w2T_r, b2T_r, msg_r):
        hT = _lrelu(jnp.dot(w1T_r[...], ea_r[...],
                            preferred_element_type=_f32) + b1_r[...])
        oT = o_r[:, 0:DIM].T
        ewT = jnp.dot(w2T_r[...], hT,
                      preferred_element_type=_f32)
        accT = jnp.dot(b2T_r[...], oT,
                       preferred_element_type=_f32)
        for d in range(DIM):
            accT += oT[d:d + 1, :] * ewT[d * DIM:(d + 1) * DIM, :]
        msg_r[:, 0:DIM] = accT.T
        msg_r[:, DIM:W] = jnp.zeros((EB, W - DIM), _f32)

    grid = E // EB
    return pl.pallas_call(
        body,
        grid=(grid,),
        in_specs=[
            pl.BlockSpec((4, EB), lambda i: (0, i)),
            pl.BlockSpec((EB, W), lambda i: (i, 0)),
            pl.BlockSpec((128, 4), lambda i: (0, 0)),
            pl.BlockSpec((128, 1), lambda i: (0, 0)),
            pl.BlockSpec((DIM * DIM, 128), lambda i: (0, 0)),
            pl.BlockSpec((DIM, DIM), lambda i: (0, 0)),
        ],
        out_specs=pl.BlockSpec((EB, W), lambda i: (i, 0)),
        out_shape=jax.ShapeDtypeStruct((E, W), _f32),
    )(eaT, osrc, w1T, b1c, w2T, b2T)


def _tc_update(aggp, onesp, h, conv_root, conv_bias,
               w_ir, w_iz, w_in, w_hr, w_hz, w_hn, b_r, b_z, b_in, b_hn):
    def body(aggp_r, onesp_r, h_r, cr_r, cb_r,
             wir_r, wiz_r, win_r, whr_r, whz_r, whn_r,
             br_r, bz_r, bin_r, bhn_r, out_r):
        h0 = h_r[:, 0:DIM]
        cnt = onesp_r[0, :, 0:1] + onesp_r[1, :, 0:1]
        inv = 1.0 / jnp.clip(cnt, 1.0, None)
        agg = (aggp_r[0, :, 0:DIM] + aggp_r[1, :, 0:DIM]) * inv
        m = _lrelu(agg + jnp.dot(h0, cr_r[...], preferred_element_type=_f32)
                   + cb_r[...])
        r = jax.nn.sigmoid(jnp.dot(m, wir_r[...], preferred_element_type=_f32)
                           + jnp.dot(h0, whr_r[...], preferred_element_type=_f32)
                           + br_r[...])
        z = jax.nn.sigmoid(jnp.dot(m, wiz_r[...], preferred_element_type=_f32)
                           + jnp.dot(h0, whz_r[...], preferred_element_type=_f32)
                           + bz_r[...])
        hn = jnp.dot(h0, whn_r[...], preferred_element_type=_f32) + bhn_r[...]
        n = jnp.tanh(jnp.dot(m, win_r[...], preferred_element_type=_f32)
                     + bin_r[...] + r * hn)
        out_r[:, 0:DIM] = (1.0 - z) * n + z * h0
        out_r[:, DIM:W] = jnp.zeros((N, W - DIM), _f32)

    return pl.pallas_call(
        body,
        out_shape=jax.ShapeDtypeStruct((N, W), _f32),
    )(aggp, onesp, h, conv_root, conv_bias,
      w_ir, w_iz, w_in, w_hr, w_hz, w_hn, b_r, b_z, b_in, b_hn)


def _tc_post(h, lin1_w, lin1_b, batch2d,
             wq_i, wq_f, wq_g, wq_o, wh_i, wh_f, wh_g, wh_o,
             b_i, b_f, b_g, b_o, lin3_w, lin3_b,
             n2s_w1, n2s_b1, n2s_w2, n2s_b2,
             n2j_w1, n2j_b1, n2j_w2, n2j_b2,
             stem2d, jb2d, pairmat):
    def body(h_r, w1_r, b1_r, bat_r,
             wqi_r, wqf_r, wqg_r, wqo_r, whi_r, whf_r, whg_r, who_r,
             bi_r, bf_r, bg_r, bo_r, w3_r, b3_r,
             sw1_r, sb1_r, sw2_r, sb2_r, jw1_r, jb1_r, jw2_r, jb2_r,
             sidx_r, jidx_r, pm_r, sout_r, stem_r, jbm_r):
        h0 = h_r[:, 0:DIM]
        pa = _lrelu(
            jnp.dot(h0, w1_r[...], preferred_element_type=_f32) + b1_r[...])
        sh_all = _lrelu(
            jnp.dot(pa, sw1_r[...], preferred_element_type=_f32) + sb1_r[...])
        jh_all = _lrelu(
            jnp.dot(pa, jw1_r[...], preferred_element_type=_f32) + jb1_r[...])
        iota_n = lax.broadcasted_iota(jnp.int32, (256, N), 1)
        oh_s = (sidx_r[...] == iota_n).astype(_f32)
        oh_j = (jidx_r[...] == iota_n).astype(_f32)
        sh_g = jnp.dot(oh_s, sh_all, preferred_element_type=_f32)
        stem_r[...] = (jnp.dot(sh_g, sw2_r[...], preferred_element_type=_f32)
                       + sb2_r[...])
        jh_g = jnp.dot(oh_j, jh_all, preferred_element_type=_f32)
        jb = jnp.dot(jh_g, jw2_r[...], preferred_element_type=_f32) + jb2_r[...]
        jbm_r[...] = jnp.dot(pm_r[...], jb, preferred_element_type=_f32)
        ids = lax.broadcasted_iota(jnp.int32, (N, B), 1)
        mb = bat_r[...] == ids
        mm = mb.astype(_f32)
        q_star = jnp.zeros((B, 2 * DIM), _f32)
        hs = jnp.zeros((B, DIM), _f32)
        cs = jnp.zeros((B, DIM), _f32)
        for _ in range(3):
            g_i = (jnp.dot(q_star, wqi_r[...], preferred_element_type=_f32)
                   + jnp.dot(hs, whi_r[...], preferred_element_type=_f32)
                   + bi_r[...])
            g_f = (jnp.dot(q_star, wqf_r[...], preferred_element_type=_f32)
                   + jnp.dot(hs, whf_r[...], preferred_element_type=_f32)
                   + bf_r[...])
            g_g = (jnp.dot(q_star, wqg_r[...], preferred_element_type=_f32)
                   + jnp.dot(hs, whg_r[...], preferred_element_type=_f32)
                   + bg_r[...])
            g_o = (jnp.dot(q_star, wqo_r[...], preferred_element_type=_f32)
                   + jnp.dot(hs, who_r[...], preferred_element_type=_f32)
                   + bo_r[...])
            cs = jax.nn.sigmoid(g_f) * cs + jax.nn.sigmoid(g_i) * jnp.tanh(g_g)
            hs = jax.nn.sigmoid(g_o) * jnp.tanh(cs)
            qb = jnp.dot(mm, hs, preferred_element_type=_f32)
            e = jnp.sum(h0 * qb, axis=1, keepdims=True)
            masked = jnp.where(mb, e, -jnp.inf)
            emax = jnp.max(masked, axis=0, keepdims=True)
            emax = jnp.where(jnp.isfinite(emax), emax, 0.0)
            eb = jnp.sum(mm * emax, axis=1, keepdims=True)
            ee = jnp.exp(e - eb)
            denom = lax.dot_general(mm, ee, (((0,), (0,)), ((), ())),
                                    preferred_element_type=_f32)
            db = jnp.dot(mm, denom, preferred_element_type=_f32)
            a = ee / db
            r_read = lax.dot_general(mm, a * h0, (((0,), (0,)), ((), ())),
                                     preferred_element_type=_f32)
            q_star = jnp.concatenate([hs, r_read], axis=1)
        sout_r[...] = (jnp.dot(q_star, w3_r[...], preferred_element_type=_f32)
                       + b3_r[...])

    return pl.pallas_call(
        body,
        out_shape=(jax.ShapeDtypeStruct((B, 2), _f32),
                   jax.ShapeDtypeStruct((256, 105), _f32),
                   jax.ShapeDtypeStruct((B, 1), _f32)),
    )(h, lin1_w, lin1_b, batch2d,
      wq_i, wq_f, wq_g, wq_o, wh_i, wh_f, wh_g, wh_o,
      b_i, b_f, b_g, b_o, lin3_w, lin3_b,
      n2s_w1, n2s_b1, n2s_w2, n2s_b2, n2j_w1, n2j_b1, n2j_w2, n2j_b2,
      stem2d, jb2d, pairmat)


def kernel(x, edge_attr, lin0_w, lin0_b, en_w1, en_b1, en_w2, en_b2,
           conv_root, conv_bias, gru_w_ih, gru_w_hh, gru_b_ih, gru_b_hh,
           lin1_w, lin1_b, lstm_w_ih, lstm_w_hh, lstm_b_ih, lstm_b_hh,
           lin3_w, lin3_b, n2s_w1, n2s_b1, n2s_w2, n2s_b2,
           n2j_w1, n2j_b1, n2j_w2, n2j_b2,
           edge_index, batch, stem_atmidx, jbond_atmidx):
    i32 = jnp.int32
    src2 = edge_index[0].astype(i32).reshape(NW * KCH, CHUNK)
    dst2 = edge_index[1].astype(i32).reshape(NW * KCH, CHUNK)
    eaT = edge_attr.T
    w1T = en_w1.T
    b1c = en_b1.reshape(128, 1)
    w2T = en_w2.T
    b2T = en_b2.reshape(DIM, DIM).T
    zeros_n = jnp.zeros((N, W), _f32)
    ones_e = jnp.ones((E, W), _f32)

    w_ir, w_iz, w_in = (gru_w_ih[0:DIM].T, gru_w_ih[DIM:2 * DIM].T,
                        gru_w_ih[2 * DIM:3 * DIM].T)
    w_hr, w_hz, w_hn = (gru_w_hh[0:DIM].T, gru_w_hh[DIM:2 * DIM].T,
                        gru_w_hh[2 * DIM:3 * DIM].T)
    b_r = (gru_b_ih[0:DIM] + gru_b_hh[0:DIM]).reshape(1, DIM)
    b_z = (gru_b_ih[DIM:2 * DIM] + gru_b_hh[DIM:2 * DIM]).reshape(1, DIM)
    b_in = gru_b_ih[2 * DIM:3 * DIM].reshape(1, DIM)
    b_hn = gru_b_hh[2 * DIM:3 * DIM].reshape(1, DIM)

    wq_i, wq_f, wq_g, wq_o = (lstm_w_ih[0:DIM].T, lstm_w_ih[DIM:2 * DIM].T,
                              lstm_w_ih[2 * DIM:3 * DIM].T,
                              lstm_w_ih[3 * DIM:4 * DIM].T)
    wh_i, wh_f, wh_g, wh_o = (lstm_w_hh[0:DIM].T, lstm_w_hh[DIM:2 * DIM].T,
                              lstm_w_hh[2 * DIM:3 * DIM].T,
                              lstm_w_hh[3 * DIM:4 * DIM].T)
    lb = lstm_b_ih + lstm_b_hh
    b_i, b_f = lb[0:DIM].reshape(1, DIM), lb[DIM:2 * DIM].reshape(1, DIM)
    b_g, b_o = (lb[2 * DIM:3 * DIM].reshape(1, DIM),
                lb[3 * DIM:4 * DIM].reshape(1, DIM))

    h = _tc_pre(x, lin0_w, lin0_b.reshape(1, DIM))
    onesp = _sc_scatter(ones_e, dst2, zeros_n)

    for _ in range(3):
        osrc = _sc_gather(h, src2)
        msg = _tc_msg(eaT, osrc, w1T, b1c, w2T, b2T)
        aggp = _sc_scatter(msg, dst2, zeros_n)
        h = _tc_update(aggp, onesp, h, conv_root, conv_bias.reshape(1, DIM),
                       w_ir, w_iz, w_in, w_hr, w_hz, w_hn,
                       b_r, b_z, b_in, b_hn)

    cols = jnp.arange(256, dtype=i32)
    pairmat = (cols[None, :] // 2
               == jnp.arange(B, dtype=i32)[:, None]).astype(_f32) * 0.5

    sout, stem_preds, jbm = _tc_post(
        h, lin1_w, lin1_b.reshape(1, 8 * DIM), batch.astype(i32).reshape(N, 1),
        wq_i, wq_f, wq_g, wq_o, wh_i, wh_f, wh_g, wh_o,
        b_i, b_f, b_g, b_o, lin3_w, lin3_b.reshape(1, 2),
        n2s_w1, n2s_b1.reshape(1, DIM), n2s_w2, n2s_b2.reshape(1, 105),
        n2j_w1, n2j_b1.reshape(1, DIM), n2j_w2, n2j_b2.reshape(1, 1),
        stem_atmidx.astype(i32).reshape(256, 1),
        jbond_atmidx.astype(i32).reshape(256, 1), pairmat)

    return (sout, stem_preds, jbm.reshape(B))
